# baseline (speedup 1.0000x reference)
import jax
import jax.numpy as jnp
from jax import lax
from jax.experimental import pallas as pl
from jax.experimental.pallas import tpu as pltpu
from jax.experimental.pallas import tpu_sc as plsc

N_ASSETS = 10
SUB_SIZE = 100000
TOTAL_VOCAB = N_ASSETS * SUB_SIZE
EMBED_DIM = 32
BATCH = 16384
ROWS_PER_LINE = 128 // EMBED_DIM
N_LINES = TOTAL_VOCAB // ROWS_PER_LINE

_INFO = plsc.get_sparse_core_info()
_NC = _INFO.num_cores
_NS = _INFO.num_subcores
_LANES = _INFO.num_lanes
_NW = _NC * _NS
_BPW = BATCH // _NW
_CHUNK = 128
_NCHUNK = _BPW // _CHUNK
_NBUF = 2


def _sc_body(asset_hbm, shape_hbm, table_hbm, out_hbm,
             asset_v, shape_v, lidx_v, q32_v, lines0, lines1, rows_v,
             sem0, sem1):
    wid = lax.axis_index("s") * _NC + lax.axis_index("c")
    base = wid * _BPW

    pltpu.sync_copy(asset_hbm.at[pl.ds(base, _BPW)], asset_v)
    pltpu.sync_copy(shape_hbm.at[pl.ds(base, _BPW)], shape_v)

    for i in range(_BPW // _LANES):
        off = i * _LANES
        a = asset_v[pl.ds(off, _LANES)]
        s = shape_v[pl.ds(off, _LANES)]
        idx = a * SUB_SIZE + s
        lidx_v[pl.ds(off, _LANES)] = lax.shift_right_logical(idx, 2)
        q32_v[pl.ds(off, _LANES)] = (idx & (ROWS_PER_LINE - 1)) * EMBED_DIM

    bufs = (lines0, lines1)
    sems = (sem0, sem1)

    def _fire(c):
        cp = pltpu.make_async_copy(
            table_hbm.at[lidx_v.at[pl.ds(c * _CHUNK, _CHUNK)]],
            bufs[c % _NBUF],
            sems[c % _NBUF],
        )
        cp.start()
        return cp

    copies = [_fire(c) for c in range(_NBUF)]

    lane = lax.iota(jnp.int32, _LANES)
    for c in range(_NCHUNK):
        copies[c].wait()
        buf = bufs[c % _NBUF]

        for r in range(_CHUNK // _LANES):
            rowv = r * _LANES + lane
            goff = c * _CHUNK + r * _LANES
            q32 = q32_v[pl.ds(goff, _LANES)]
            dstv = goff + lane

            def _extract_col(d, _, rowv=rowv, q32=q32, dstv=dstv, buf=buf):
                val = plsc.load_gather(buf, [rowv, q32 + d])
                plsc.store_scatter(rows_v, [dstv, lane * 0 + d], val)
                return 0

            lax.fori_loop(0, EMBED_DIM, _extract_col, 0, unroll=4)

        if c + _NBUF < _NCHUNK:
            copies.append(_fire(c + _NBUF))

    pltpu.sync_copy(rows_v, out_hbm.at[pl.ds(base, _BPW)])


@jax.jit
def _lookup(asset_index, shape_index, table_lines):
    mesh = plsc.VectorSubcoreMesh(core_axis_name="c", subcore_axis_name="s")
    fn = pl.kernel(
        _sc_body,
        out_type=jax.ShapeDtypeStruct((BATCH, EMBED_DIM), jnp.float32),
        mesh=mesh,
        scratch_types=[
            pltpu.VMEM((_BPW,), jnp.int32),
            pltpu.VMEM((_BPW,), jnp.int32),
            pltpu.VMEM((_BPW,), jnp.int32),
            pltpu.VMEM((_BPW,), jnp.int32),
            pltpu.VMEM((_CHUNK, 128), jnp.float32),
            pltpu.VMEM((_CHUNK, 128), jnp.float32),
            pltpu.VMEM((_BPW, EMBED_DIM), jnp.float32),
            pltpu.SemaphoreType.DMA,
            pltpu.SemaphoreType.DMA,
        ],
        compiler_params=pltpu.CompilerParams(needs_layout_passes=False),
    )
    return fn(asset_index, shape_index, table_lines)


def kernel(asset_index, shape_index, table):
    table_lines = table.reshape(N_LINES, ROWS_PER_LINE * EMBED_DIM)
    return _lookup(asset_index.astype(jnp.int32),
                   shape_index.astype(jnp.int32),
                   table_lines)


# speedup vs baseline: 3.5921x; 3.5921x over previous
import jax
import jax.numpy as jnp
from jax import lax
from jax.experimental import pallas as pl
from jax.experimental.pallas import tpu as pltpu
from jax.experimental.pallas import tpu_sc as plsc

N_ASSETS = 10
SUB_SIZE = 100000
TOTAL_VOCAB = N_ASSETS * SUB_SIZE
EMBED_DIM = 32
BATCH = 16384

_INFO = plsc.get_sparse_core_info()
_NC = _INFO.num_cores
_NS = _INFO.num_subcores
_LANES = _INFO.num_lanes
_NW = _NC * _NS
_BPW = BATCH // _NW
_NBLK = _BPW // _LANES

_LAST_TILE = (TOTAL_VOCAB // 128) - 1
_TAIL_START = (_LAST_TILE + 1) * 128
_TAIL_LEN = TOTAL_VOCAB - _TAIL_START


def _sc_body(asset_hbm, shape_hbm, tt_hbm, tail_hbm, outt_hbm,
             asset_v, shape_v, idx_v, slots_v, tail_v, rows_v, sem):
    wid = lax.axis_index("s") * _NC + lax.axis_index("c")
    base = wid * _BPW

    pltpu.sync_copy(asset_hbm.at[pl.ds(base, _BPW)], asset_v)
    pltpu.sync_copy(shape_hbm.at[pl.ds(base, _BPW)], shape_v)
    pltpu.sync_copy(tail_hbm, tail_v)

    for i in range(_NBLK):
        off = i * _LANES
        a = asset_v[pl.ds(off, _LANES)]
        s = shape_v[pl.ds(off, _LANES)]
        idx_v[pl.ds(off, _LANES)] = a * SUB_SIZE + s

    lane = lax.iota(jnp.int32, _LANES)

    def _block(b, _):
        vec = idx_v[pl.ds(b * _LANES, _LANES)]
        copies = []
        for l in range(_LANES):
            jc = jnp.minimum(lax.shift_right_logical(vec[l], 7),
                             jnp.int32(_LAST_TILE))
            wstart = pl.multiple_of(jc * 128, 128)
            cp = pltpu.make_async_copy(
                tt_hbm.at[:, pl.ds(wstart, 128)],
                slots_v.at[l],
                sem,
            )
            cp.start()
            copies.append(cp)
        for cp in copies:
            cp.wait()
        for l in range(_LANES):
            i = vec[l]
            cvec = lane * 0 + (i & 127)
            jvec = lane * 0 + (b * _LANES + l)
            is_tail = i >= _TAIL_START
            trow = lane * 0 + jnp.maximum(i - _TAIL_START, 0)
            slot = slots_v.at[l]
            for h in range(EMBED_DIM // _LANES):
                e16 = h * _LANES + lane
                vm = plsc.load_gather(slot, [e16, cvec])
                vt = plsc.load_gather(tail_v, [trow, e16])
                val = jnp.where(is_tail, vt, vm)
                plsc.store_scatter(rows_v, [e16, jvec], val)
        return 0

    lax.fori_loop(0, _NBLK, _block, 0)

    pltpu.sync_copy(rows_v, outt_hbm.at[:, pl.ds(base, _BPW)])


@jax.jit
def _lookup(asset_index, shape_index, table_t, tail):
    mesh = plsc.VectorSubcoreMesh(core_axis_name="c", subcore_axis_name="s")
    fn = pl.kernel(
        _sc_body,
        out_type=jax.ShapeDtypeStruct((EMBED_DIM, BATCH), jnp.float32),
        mesh=mesh,
        scratch_types=[
            pltpu.VMEM((_BPW,), jnp.int32),
            pltpu.VMEM((_BPW,), jnp.int32),
            pltpu.VMEM((_BPW,), jnp.int32),
            pltpu.VMEM((_LANES, EMBED_DIM, 128), jnp.float32),
            pltpu.VMEM((_TAIL_LEN, EMBED_DIM), jnp.float32),
            pltpu.VMEM((EMBED_DIM, _BPW), jnp.float32),
            pltpu.SemaphoreType.DMA,
        ],
        compiler_params=pltpu.CompilerParams(needs_layout_passes=False),
    )
    return fn(asset_index, shape_index, table_t, tail)


def kernel(asset_index, shape_index, table):
    tail = table[_TAIL_START:, :]
    out_t = _lookup(asset_index.astype(jnp.int32),
                    shape_index.astype(jnp.int32),
                    table.T, tail)
    return out_t.T


# speedup vs baseline: 3.8602x; 1.0746x over previous
import jax
import jax.numpy as jnp
from jax import lax
from jax.experimental import pallas as pl
from jax.experimental.pallas import tpu as pltpu
from jax.experimental.pallas import tpu_sc as plsc

N_ASSETS = 10
SUB_SIZE = 100000
TOTAL_VOCAB = N_ASSETS * SUB_SIZE
EMBED_DIM = 32
BATCH = 16384

_INFO = plsc.get_sparse_core_info()
_NC = _INFO.num_cores
_NS = _INFO.num_subcores
_LANES = _INFO.num_lanes
_NW = _NC * _NS
_BPW = BATCH // _NW
_NBLK = _BPW // _LANES

_LAST_TILE = (TOTAL_VOCAB // 128) - 1
_TAIL_START = (_LAST_TILE + 1) * 128
_TAIL_LEN = TOTAL_VOCAB - _TAIL_START


_G = 8
_NG = _BPW // _G


def _sc_body(asset_hbm, shape_hbm, tt_hbm, tail_hbm, outt_hbm,
             asset_v, shape_v, idx_v, slots_v, tail_v, rows_v, sem_a, sem_b):
    wid = lax.axis_index("s") * _NC + lax.axis_index("c")
    base = wid * _BPW

    pltpu.sync_copy(asset_hbm.at[pl.ds(base, _BPW)], asset_v)
    pltpu.sync_copy(shape_hbm.at[pl.ds(base, _BPW)], shape_v)
    pltpu.sync_copy(tail_hbm, tail_v)

    lane = lax.iota(jnp.int32, _LANES)

    for i in range(_NBLK):
        off = i * _LANES
        a = asset_v[pl.ds(off, _LANES)]
        s = shape_v[pl.ds(off, _LANES)]
        idx_v[pl.ds(off, _LANES)] = a * SUB_SIZE + s
    idx_v[pl.ds(_BPW, _LANES)] = lane * 0

    def _fire(vec, lo, slot_base, sem):
        for l in range(_G):
            jc = jnp.minimum(lax.shift_right_logical(vec[lo + l], 7),
                             jnp.int32(_LAST_TILE))
            wstart = pl.multiple_of(jc * 128, 128)
            pltpu.make_async_copy(
                tt_hbm.at[:, pl.ds(wstart, 128)],
                slots_v.at[slot_base + l],
                sem,
            ).start()

    def _drain(slot_base, sem):
        for l in range(_G):
            pltpu.make_async_copy(
                tt_hbm.at[:, pl.ds(0, 128)],
                slots_v.at[slot_base + l],
                sem,
            ).wait()

    def _extract(vec, lo, slot_base, g):
        for l in range(_G):
            i = vec[lo + l]
            cvec = lane * 0 + (i & 127)
            jvec = lane * 0 + (g * _G + l)
            is_tail = i >= _TAIL_START
            trow = lane * 0 + jnp.maximum(i - _TAIL_START, 0)
            slot = slots_v.at[slot_base + l]
            for h in range(EMBED_DIM // _LANES):
                e16 = h * _LANES + lane
                vm = plsc.load_gather(slot, [e16, cvec])
                vt = plsc.load_gather(tail_v, [trow, e16])
                val = jnp.where(is_tail, vt, vm)
                plsc.store_scatter(rows_v, [e16, jvec], val)

    vec0 = idx_v[pl.ds(0, _LANES)]
    _fire(vec0, 0, 0, sem_a)
    _fire(vec0, _G, _G, sem_b)

    def _pair(t, _):
        vec = idx_v[pl.ds(t * _LANES, _LANES)]
        vec_n = idx_v[pl.ds(t * _LANES + _LANES, _LANES)]
        _drain(0, sem_a)
        _extract(vec, 0, 0, 2 * t)

        @pl.when(t < _NG // 2 - 1)
        def _():
            _fire(vec_n, 0, 0, sem_a)

        _drain(_G, sem_b)
        _extract(vec, _G, _G, 2 * t + 1)

        @pl.when(t < _NG // 2 - 1)
        def _():
            _fire(vec_n, _G, _G, sem_b)

        return 0

    lax.fori_loop(0, _NG // 2, _pair, 0)

    pltpu.sync_copy(rows_v, outt_hbm.at[:, pl.ds(base, _BPW)])


@jax.jit
def _lookup(asset_index, shape_index, table_t, tail):
    mesh = plsc.VectorSubcoreMesh(core_axis_name="c", subcore_axis_name="s")
    fn = pl.kernel(
        _sc_body,
        out_type=jax.ShapeDtypeStruct((EMBED_DIM, BATCH), jnp.float32),
        mesh=mesh,
        scratch_types=[
            pltpu.VMEM((_BPW,), jnp.int32),
            pltpu.VMEM((_BPW,), jnp.int32),
            pltpu.VMEM((_BPW + _LANES,), jnp.int32),
            pltpu.VMEM((_LANES, EMBED_DIM, 128), jnp.float32),
            pltpu.VMEM((_TAIL_LEN, EMBED_DIM), jnp.float32),
            pltpu.VMEM((EMBED_DIM, _BPW), jnp.float32),
            pltpu.SemaphoreType.DMA,
            pltpu.SemaphoreType.DMA,
        ],
        compiler_params=pltpu.CompilerParams(needs_layout_passes=False),
    )
    return fn(asset_index, shape_index, table_t, tail)


def kernel(asset_index, shape_index, table):
    tail = table[_TAIL_START:, :]
    out_t = _lookup(asset_index.astype(jnp.int32),
                    shape_index.astype(jnp.int32),
                    table.T, tail)
    return out_t.T


# speedup vs baseline: 4.1510x; 1.0754x over previous
import jax
import jax.numpy as jnp
from jax import lax
from jax.experimental import pallas as pl
from jax.experimental.pallas import tpu as pltpu
from jax.experimental.pallas import tpu_sc as plsc

N_ASSETS = 10
SUB_SIZE = 100000
TOTAL_VOCAB = N_ASSETS * SUB_SIZE
EMBED_DIM = 32
BATCH = 16384

_INFO = plsc.get_sparse_core_info()
_NC = _INFO.num_cores
_NS = _INFO.num_subcores
_LANES = _INFO.num_lanes
_NW = _NC * _NS
_BPW = BATCH // _NW
_NBLK = _BPW // _LANES

_LAST_TILE = (TOTAL_VOCAB // 128) - 1
_TAIL_START = (_LAST_TILE + 1) * 128
_TAIL_LEN = TOTAL_VOCAB - _TAIL_START


_G = 8
_NG = _BPW // _G


def _sc_body(asset_hbm, shape_hbm, tt_hbm, tail_hbm, outt_hbm,
             asset_v, shape_v, idx_v, jc_v, slots_v, tail_v, rows_v,
             sem_a, sem_b, sem_c):
    wid = lax.axis_index("s") * _NC + lax.axis_index("c")
    base = wid * _BPW

    pltpu.sync_copy(asset_hbm.at[pl.ds(base, _BPW)], asset_v)
    pltpu.sync_copy(shape_hbm.at[pl.ds(base, _BPW)], shape_v)
    pltpu.sync_copy(tail_hbm, tail_v)

    lane = lax.iota(jnp.int32, _LANES)

    for i in range(_NBLK):
        off = i * _LANES
        a = asset_v[pl.ds(off, _LANES)]
        s = shape_v[pl.ds(off, _LANES)]
        idx = a * SUB_SIZE + s
        idx_v[pl.ds(off, _LANES)] = idx
        jc_v[pl.ds(off, _LANES)] = jnp.minimum(
            lax.shift_right_logical(idx, 7), jnp.int32(_LAST_TILE))
    idx_v[pl.ds(_BPW, _LANES)] = lane * 0
    idx_v[pl.ds(_BPW + _LANES, _LANES)] = lane * 0
    jc_v[pl.ds(_BPW, _LANES)] = lane * 0
    jc_v[pl.ds(_BPW + _LANES, _LANES)] = lane * 0

    def _fire(g, slot_base, sem):
        jcs = jc_v[pl.ds(g * _G, _LANES)]
        for l in range(_G):
            wstart = pl.multiple_of(jcs[l] * 128, 128)
            pltpu.make_async_copy(
                tt_hbm.at[:, pl.ds(wstart, 128)],
                slots_v.at[slot_base + l],
                sem,
            ).start()

    def _drain(slot_base, sem):
        for l in range(_G):
            pltpu.make_async_copy(
                tt_hbm.at[:, pl.ds(0, 128)],
                slots_v.at[slot_base + l],
                sem,
            ).wait()

    def _extract(g, slot_base):
        vec = idx_v[pl.ds(g * _G, _LANES)]
        for l in range(_G):
            i = vec[l]
            cvec = lane * 0 + (i & 127)
            jvec = lane * 0 + (g * _G + l)
            is_tail = i >= _TAIL_START
            trow = lane * 0 + jnp.maximum(i - _TAIL_START, 0)
            slot = slots_v.at[slot_base + l]
            for h in range(EMBED_DIM // _LANES):
                e16 = h * _LANES + lane
                vm = plsc.load_gather(slot, [e16, cvec])
                vt = plsc.load_gather(tail_v, [trow, e16])
                val = jnp.where(is_tail, vt, vm)
                plsc.store_scatter(rows_v, [e16, jvec], val)

    _fire(0, 0, sem_a)
    _fire(1, _G, sem_b)
    _fire(2, 2 * _G, sem_c)

    def _triple(t, _):
        g = 3 * t
        for ph, sem in ((0, sem_a), (1, sem_b), (2, sem_c)):
            _drain(ph * _G, sem)
            _extract(g + ph, ph * _G)

            @pl.when(g + ph + 3 < _NG)
            def _(ph=ph, sem=sem):
                _fire(g + ph + 3, ph * _G, sem)

        return 0

    lax.fori_loop(0, _NG // 3, _triple, 0)

    for g in range((_NG // 3) * 3, _NG):
        ph = g % 3
        sem = (sem_a, sem_b, sem_c)[ph]
        _drain(ph * _G, sem)
        _extract(g, ph * _G)

    pltpu.sync_copy(rows_v, outt_hbm.at[:, pl.ds(base, _BPW)])


@jax.jit
def _lookup(asset_index, shape_index, table_t, tail):
    mesh = plsc.VectorSubcoreMesh(core_axis_name="c", subcore_axis_name="s")
    fn = pl.kernel(
        _sc_body,
        out_type=jax.ShapeDtypeStruct((EMBED_DIM, BATCH), jnp.float32),
        mesh=mesh,
        scratch_types=[
            pltpu.VMEM((_BPW,), jnp.int32),
            pltpu.VMEM((_BPW,), jnp.int32),
            pltpu.VMEM((_BPW + 2 * _LANES,), jnp.int32),
            pltpu.VMEM((_BPW + 2 * _LANES,), jnp.int32),
            pltpu.VMEM((3 * _G, EMBED_DIM, 128), jnp.float32),
            pltpu.VMEM((_TAIL_LEN, EMBED_DIM), jnp.float32),
            pltpu.VMEM((EMBED_DIM, _BPW), jnp.float32),
            pltpu.SemaphoreType.DMA,
            pltpu.SemaphoreType.DMA,
            pltpu.SemaphoreType.DMA,
        ],
        compiler_params=pltpu.CompilerParams(needs_layout_passes=False),
    )
    return fn(asset_index, shape_index, table_t, tail)


def kernel(asset_index, shape_index, table):
    tail = table[_TAIL_START:, :]
    out_t = _lookup(asset_index.astype(jnp.int32),
                    shape_index.astype(jnp.int32),
                    table.T, tail)
    return out_t.T
